# baseline (speedup 1.0000x reference)
import functools

import jax
import jax.numpy as jnp
from jax import lax
from jax.experimental import pallas as pl
from jax.experimental.pallas import tpu as pltpu
from jax.experimental.pallas import tpu_sc as plsc

DIM = 16
ALPHA = 0.001
LANES = 16
CHUNK = 128


def _dot_kernel(n_batch, n_workers, ij_hbm, p_hbm, m_hbm, out_hbm,
                ij_v, si_v, sj_v, oi_v, oj_v, out_v,
                p_buf0, m_buf0, p_buf1, m_buf1, sem0, sem1):
    bpw = n_batch // n_workers
    wid = lax.axis_index("s") * 2 + lax.axis_index("c")
    base = wid * bpw

    pltpu.sync_copy(ij_hbm.at[pl.ds(2 * base, 2 * bpw)], ij_v)

    def unzip_block(blk, _):
        b0 = blk * LANES
        flat = 2 * (b0 + lax.iota(jnp.int32, LANES))
        iv = plsc.load_gather(ij_v, [flat])
        jv = plsc.load_gather(ij_v, [flat + 1])
        si_v[pl.ds(b0, LANES)] = lax.shift_right_logical(iv, 3)
        sj_v[pl.ds(b0, LANES)] = lax.shift_right_logical(jv, 3)
        oi_v[pl.ds(b0, LANES)] = (iv & 7) * DIM
        oj_v[pl.ds(b0, LANES)] = (jv & 7) * DIM
        return 0

    lax.fori_loop(0, bpw // LANES, unzip_block, 0, unroll=4)

    p_bufs = (p_buf0, p_buf1)
    m_bufs = (m_buf0, m_buf1)
    sems = (sem0, sem1)
    n_chunks = bpw // CHUNK

    def fire(c):
        slot = c % 2
        s = pl.ds(c * CHUNK, CHUNK)
        cp = pltpu.async_copy(p_hbm.at[si_v.at[s]], p_bufs[slot], sems[slot])
        cm = pltpu.async_copy(m_hbm.at[sj_v.at[s]], m_bufs[slot], sems[slot])
        return cp, cm

    def compute(c):
        slot = c % 2
        pbuf = p_bufs[slot]
        mbuf = m_bufs[slot]

        def dot_block(blk, _):
            rows = blk * LANES + lax.iota(jnp.int32, LANES)
            b0g = c * CHUNK + blk * LANES
            coli = oi_v[pl.ds(b0g, LANES)]
            colj = oj_v[pl.ds(b0g, LANES)]
            acc = jnp.zeros((LANES,), jnp.float32)
            for d in range(DIM):
                pv = plsc.load_gather(pbuf, [rows, coli + d])
                mv = plsc.load_gather(mbuf, [rows, colj + d])
                acc = acc + pv * mv
            out_v[pl.ds(b0g, LANES)] = acc * jnp.float32(ALPHA)
            return 0

        lax.fori_loop(0, CHUNK // LANES, dot_block, 0)

    inflight = fire(0)
    for c in range(n_chunks):
        nxt = fire(c + 1) if c + 1 < n_chunks else None
        for cp in inflight:
            cp.wait()
        compute(c)
        inflight = nxt

    pltpu.sync_copy(out_v, out_hbm.at[pl.ds(base, bpw)])


def kernel(ij, P, M):
    ij_flat = ij.astype(jnp.int32).reshape(-1)
    p_sup = P.reshape(P.shape[0] // 8, 8 * DIM)
    m_sup = M.reshape(M.shape[0] // 8, 8 * DIM)
    n_batch = ij.shape[0]
    info = plsc.get_sparse_core_info()
    n_workers = info.num_cores * info.num_subcores
    bpw = n_batch // n_workers

    mesh = plsc.VectorSubcoreMesh(core_axis_name="c", subcore_axis_name="s")
    run = pl.kernel(
        functools.partial(_dot_kernel, n_batch, n_workers),
        out_type=jax.ShapeDtypeStruct((n_batch,), jnp.float32),
        mesh=mesh,
        scratch_types=[
            pltpu.VMEM((2 * bpw,), jnp.int32),
            pltpu.VMEM((bpw,), jnp.int32),
            pltpu.VMEM((bpw,), jnp.int32),
            pltpu.VMEM((bpw,), jnp.int32),
            pltpu.VMEM((bpw,), jnp.int32),
            pltpu.VMEM((bpw,), jnp.float32),
            pltpu.VMEM((CHUNK, 8 * DIM), jnp.float32),
            pltpu.VMEM((CHUNK, 8 * DIM), jnp.float32),
            pltpu.VMEM((CHUNK, 8 * DIM), jnp.float32),
            pltpu.VMEM((CHUNK, 8 * DIM), jnp.float32),
            pltpu.SemaphoreType.DMA,
            pltpu.SemaphoreType.DMA,
        ],
        compiler_params=pltpu.CompilerParams(needs_layout_passes=False),
    )
    return run(ij_flat, p_sup, m_sup)


# speedup vs baseline: 4.9090x; 4.9090x over previous
import functools

import jax
import jax.numpy as jnp
from jax import lax
from jax.experimental import pallas as pl
from jax.experimental.pallas import tpu as pltpu
from jax.experimental.pallas import tpu_sc as plsc

DIM = 16
ALPHA = 0.001
LANES = 16
WIN = 128
HALF = 8


def _dot_kernel(n_batch, n_rows, n_workers, ij_hbm, pt_hbm, mt_hbm, out_hbm,
                ij_v, iv_v, jv_v, out_v, pw, mw, sem_p, sem_m):
    bpw = n_batch // n_workers
    n_blocks = bpw // LANES
    wid = lax.axis_index("s") * 2 + lax.axis_index("c")
    base = wid * bpw

    pltpu.sync_copy(ij_hbm.at[pl.ds(2 * base, 2 * bpw)], ij_v)

    def unzip_block(blk, _):
        b0 = blk * LANES
        flat = 2 * (b0 + lax.iota(jnp.int32, LANES))
        iv_v[pl.ds(b0, LANES)] = plsc.load_gather(ij_v, [flat])
        jv_v[pl.ds(b0, LANES)] = plsc.load_gather(ij_v, [flat + 1])
        return 0

    lax.fori_loop(0, n_blocks, unzip_block, 0, unroll=4)

    def win_off(idx_scalar):
        return pl.multiple_of((idx_scalar >> 7) << 7, WIN)

    rows = lax.iota(jnp.int32, LANES)

    def block_body(blk, _):
        b0 = blk * LANES
        iv = iv_v[pl.ds(b0, LANES)]
        jv = jv_v[pl.ds(b0, LANES)]
        res = jnp.zeros((LANES,), jnp.float32)
        for half in range(2):
            lanes = range(half * HALF, (half + 1) * HALF)
            copies = []
            for l in lanes:
                u = l - half * HALF
                ci = win_off(iv[l])
                cj = win_off(jv[l])
                copies.append(pltpu.async_copy(
                    pt_hbm.at[:, pl.ds(ci, WIN)], pw.at[u], sem_p))
                copies.append(pltpu.async_copy(
                    mt_hbm.at[:, pl.ds(cj, WIN)], mw.at[u], sem_m))
            for cp in copies:
                cp.wait()
            for l in lanes:
                u = l - half * HALF
                oi = iv[l] - win_off(iv[l])
                oj = jv[l] - win_off(jv[l])
                pvec = plsc.load_gather(
                    pw.at[u], [rows, jnp.broadcast_to(oi, (LANES,))])
                mvec = plsc.load_gather(
                    mw.at[u], [rows, jnp.broadcast_to(oj, (LANES,))])
                s = jnp.sum(pvec * mvec)
                res = jnp.where(rows == l, s, res)
        out_v[pl.ds(b0, LANES)] = res * jnp.float32(ALPHA)
        return 0

    lax.fori_loop(0, n_blocks, block_body, 0)

    pltpu.sync_copy(out_v, out_hbm.at[pl.ds(base, bpw)])


def kernel(ij, P, M):
    ij_flat = ij.astype(jnp.int32).reshape(-1)
    pt = P.T
    mt = M.T
    n_batch = ij.shape[0]
    n_rows = P.shape[0]
    info = plsc.get_sparse_core_info()
    n_workers = info.num_cores * info.num_subcores
    bpw = n_batch // n_workers

    mesh = plsc.VectorSubcoreMesh(core_axis_name="c", subcore_axis_name="s")
    run = pl.kernel(
        functools.partial(_dot_kernel, n_batch, n_rows, n_workers),
        out_type=jax.ShapeDtypeStruct((n_batch,), jnp.float32),
        mesh=mesh,
        scratch_types=[
            pltpu.VMEM((2 * bpw,), jnp.int32),
            pltpu.VMEM((bpw,), jnp.int32),
            pltpu.VMEM((bpw,), jnp.int32),
            pltpu.VMEM((bpw,), jnp.float32),
            pltpu.VMEM((HALF, DIM, WIN), jnp.float32),
            pltpu.VMEM((HALF, DIM, WIN), jnp.float32),
            pltpu.SemaphoreType.DMA,
            pltpu.SemaphoreType.DMA,
        ],
        compiler_params=pltpu.CompilerParams(needs_layout_passes=False),
    )
    return run(ij_flat, pt, mt)


# speedup vs baseline: 5.9458x; 1.2112x over previous
import functools

import jax
import jax.numpy as jnp
from jax import lax
from jax.experimental import pallas as pl
from jax.experimental.pallas import tpu as pltpu
from jax.experimental.pallas import tpu_sc as plsc

DIM = 16
ALPHA = 0.001
LANES = 16
WIN = 128
HALF = 8


def _dot_kernel(n_batch, n_rows, n_workers, ij_hbm, pt_hbm, mt_hbm, out_hbm,
                ij_v, iv_v, jv_v, out_v, pw, mw,
                sem_pa, sem_ma, sem_pb, sem_mb):
    bpw = n_batch // n_workers
    n_blocks = bpw // LANES
    wid = lax.axis_index("s") * 2 + lax.axis_index("c")
    base = wid * bpw

    pltpu.sync_copy(ij_hbm.at[pl.ds(2 * base, 2 * bpw)], ij_v)

    def unzip_block(blk, _):
        b0 = blk * LANES
        flat = 2 * (b0 + lax.iota(jnp.int32, LANES))
        iv_v[pl.ds(b0, LANES)] = plsc.load_gather(ij_v, [flat])
        jv_v[pl.ds(b0, LANES)] = plsc.load_gather(ij_v, [flat + 1])
        return 0

    lax.fori_loop(0, n_blocks, unzip_block, 0, unroll=4)

    def win_off(idx_scalar):
        return pl.multiple_of((idx_scalar >> 7) << 7, WIN)

    rows = lax.iota(jnp.int32, LANES)

    def fire_half(b0, off, slot, sp, sm):
        iv = iv_v[pl.ds(b0, LANES)]
        jv = jv_v[pl.ds(b0, LANES)]
        copies = []
        for u in range(HALF):
            l = off + u
            ci = win_off(iv[l])
            cj = win_off(jv[l])
            copies.append(pltpu.async_copy(
                pt_hbm.at[:, pl.ds(ci, WIN)], pw.at[slot * HALF + u], sp))
            copies.append(pltpu.async_copy(
                mt_hbm.at[:, pl.ds(cj, WIN)], mw.at[slot * HALF + u], sm))
        return copies

    def drain_half(slot, sp, sm):
        for u in range(HALF):
            pltpu.make_async_copy(
                pt_hbm.at[:, pl.ds(0, WIN)], pw.at[slot * HALF + u], sp).wait()
            pltpu.make_async_copy(
                mt_hbm.at[:, pl.ds(0, WIN)], mw.at[slot * HALF + u], sm).wait()

    def compute_half(b0, off, slot, res):
        iv = iv_v[pl.ds(b0, LANES)]
        jv = jv_v[pl.ds(b0, LANES)]
        for u in range(HALF):
            l = off + u
            oi = iv[l] - win_off(iv[l])
            oj = jv[l] - win_off(jv[l])
            pvec = plsc.load_gather(
                pw.at[slot * HALF + u], [rows, jnp.broadcast_to(oi, (LANES,))])
            mvec = plsc.load_gather(
                mw.at[slot * HALF + u], [rows, jnp.broadcast_to(oj, (LANES,))])
            s = jnp.sum(pvec * mvec)
            res = jnp.where(rows == l, s, res)
        return res

    def block_body(blk, _):
        b0 = blk * LANES
        copies_b = fire_half(b0, HALF, 1, sem_pb, sem_mb)
        drain_half(0, sem_pa, sem_ma)
        res = compute_half(b0, 0, 0, jnp.zeros((LANES,), jnp.float32))

        @pl.when(blk + 1 < n_blocks)
        def _():
            fire_half(b0 + LANES, 0, 0, sem_pa, sem_ma)

        for cp in copies_b:
            cp.wait()
        res = compute_half(b0, HALF, 1, res)
        out_v[pl.ds(b0, LANES)] = res * jnp.float32(ALPHA)
        return 0

    fire_half(0, 0, 0, sem_pa, sem_ma)
    lax.fori_loop(0, n_blocks, block_body, 0)

    pltpu.sync_copy(out_v, out_hbm.at[pl.ds(base, bpw)])


def kernel(ij, P, M):
    ij_flat = ij.astype(jnp.int32).reshape(-1)
    pt = P.T
    mt = M.T
    n_batch = ij.shape[0]
    n_rows = P.shape[0]
    info = plsc.get_sparse_core_info()
    n_workers = info.num_cores * info.num_subcores
    bpw = n_batch // n_workers

    mesh = plsc.VectorSubcoreMesh(core_axis_name="c", subcore_axis_name="s")
    run = pl.kernel(
        functools.partial(_dot_kernel, n_batch, n_rows, n_workers),
        out_type=jax.ShapeDtypeStruct((n_batch,), jnp.float32),
        mesh=mesh,
        scratch_types=[
            pltpu.VMEM((2 * bpw,), jnp.int32),
            pltpu.VMEM((bpw,), jnp.int32),
            pltpu.VMEM((bpw,), jnp.int32),
            pltpu.VMEM((bpw,), jnp.float32),
            pltpu.VMEM((2 * HALF, DIM, WIN), jnp.float32),
            pltpu.VMEM((2 * HALF, DIM, WIN), jnp.float32),
            pltpu.SemaphoreType.DMA,
            pltpu.SemaphoreType.DMA,
            pltpu.SemaphoreType.DMA,
            pltpu.SemaphoreType.DMA,
        ],
        compiler_params=pltpu.CompilerParams(needs_layout_passes=False),
    )
    return run(ij_flat, pt, mt)
